# baseline (speedup 1.0000x reference)
import jax
import jax.numpy as jnp
from jax.experimental import pallas as pl

LATENT_DIM = 64
NUM_EMBED = 1024
RVQ_LAYERS = 4
BETA = 0.25

TILE = 512


def _conv(x, w, b, stride):
    y = jax.lax.conv_general_dilated(
        x, w, window_strides=(stride, stride), padding='SAME',
        dimension_numbers=('NHWC', 'HWIO', 'NHWC'))
    return y + b


def _conv_transpose(x, w, b, stride):
    y = jax.lax.conv_transpose(
        x, w, strides=(stride, stride), padding='SAME',
        dimension_numbers=('NHWC', 'HWIO', 'NHWC'))
    return y + b


def _rvq_kernel(z_ref, emb_ref, embt_hi_ref, embt_mid_ref, embt_lo_ref,
                zq_ref, loss_ref):
    i = pl.program_id(0)
    z = z_ref[...]
    q = jnp.zeros_like(z)
    r = z
    for layer in range(RVQ_LAYERS):
        emb = emb_ref[layer]
        sim = jnp.dot(r, emb, preferred_element_type=jnp.float32)
        rn = jnp.sum(r * r, axis=1, keepdims=True)
        en = jnp.sum(emb * emb, axis=0, keepdims=True)
        dist = rn + en - 2.0 * sim
        m = jnp.min(dist, axis=1, keepdims=True)
        iota = jax.lax.broadcasted_iota(jnp.int32, (TILE, NUM_EMBED), 1)
        cand = jnp.where(dist == m, iota, NUM_EMBED)
        idx = jnp.min(cand, axis=1, keepdims=True)
        onehot = (iota == idx).astype(jnp.float32)
        up_hi = jnp.dot(onehot, embt_hi_ref[layer],
                        preferred_element_type=jnp.float32)
        up_mid = jnp.dot(onehot, embt_mid_ref[layer],
                         preferred_element_type=jnp.float32)
        up_lo = jnp.dot(onehot, embt_lo_ref[layer],
                        preferred_element_type=jnp.float32)
        upd = (up_hi + up_mid) + up_lo
        q = q + upd
        r = r - upd
    zq_ref[...] = q
    partial = jnp.sum(r * r).reshape(1, 1)

    @pl.when(i == 0)
    def _init():
        loss_ref[...] = partial

    @pl.when(i != 0)
    def _acc():
        loss_ref[...] += partial


def _fused_rvq(z_flat, embeddings):
    n = z_flat.shape[0]
    grid = n // TILE
    emb_l = jnp.transpose(embeddings, (2, 0, 1))
    embt_l = jnp.transpose(embeddings, (2, 1, 0))
    _trunc = lambda t: jax.lax.bitcast_convert_type(
        jax.lax.bitcast_convert_type(t, jnp.uint32) & jnp.uint32(0xFFFF0000),
        jnp.float32)
    embt_hi = _trunc(embt_l)
    rem = embt_l - embt_hi
    embt_mid = _trunc(rem)
    embt_lo = rem - embt_mid

    zq, loss = pl.pallas_call(
        _rvq_kernel,
        grid=(grid,),
        in_specs=[
            pl.BlockSpec((TILE, LATENT_DIM), lambda i: (i, 0)),
            pl.BlockSpec((RVQ_LAYERS, LATENT_DIM, NUM_EMBED),
                         lambda i: (0, 0, 0)),
            pl.BlockSpec((RVQ_LAYERS, NUM_EMBED, LATENT_DIM),
                         lambda i: (0, 0, 0)),
            pl.BlockSpec((RVQ_LAYERS, NUM_EMBED, LATENT_DIM),
                         lambda i: (0, 0, 0)),
            pl.BlockSpec((RVQ_LAYERS, NUM_EMBED, LATENT_DIM),
                         lambda i: (0, 0, 0)),
        ],
        out_specs=[
            pl.BlockSpec((TILE, LATENT_DIM), lambda i: (i, 0)),
            pl.BlockSpec((1, 1), lambda i: (0, 0)),
        ],
        out_shape=[
            jax.ShapeDtypeStruct((n, LATENT_DIM), jnp.float32),
            jax.ShapeDtypeStruct((1, 1), jnp.float32),
        ],
    )(z_flat, emb_l, embt_hi, embt_mid, embt_lo)
    vq_loss = BETA * loss[0, 0] / (n * LATENT_DIM)
    return zq, vq_loss


def kernel(x, enc_w1, enc_b1, enc_w2, enc_b2, enc_w3, enc_b3,
           dec_w1, dec_b1, dec_w2, dec_b2, dec_w3, dec_b3, embeddings):
    h = jax.nn.relu(_conv(x, enc_w1, enc_b1, 2))
    h = jax.nn.relu(_conv(h, enc_w2, enc_b2, 2))
    z = _conv(h, enc_w3, enc_b3, 1)
    b, hh, ww, _ = z.shape
    zq_flat, vq_loss = _fused_rvq(z.reshape(-1, LATENT_DIM), embeddings)
    zq = zq_flat.reshape(b, hh, ww, LATENT_DIM)
    d = jax.nn.relu(_conv_transpose(zq, dec_w1, dec_b1, 2))
    d = jax.nn.relu(_conv_transpose(d, dec_w2, dec_b2, 2))
    recon = _conv_transpose(d, dec_w3, dec_b3, 1)
    return recon, vq_loss


# speedup vs baseline: 1.0526x; 1.0526x over previous
import jax
import jax.numpy as jnp
from jax.experimental import pallas as pl

LATENT_DIM = 64
NUM_EMBED = 1024
RVQ_LAYERS = 4
BETA = 0.25

TILE = 1024


def _conv(x, w, b, stride):
    y = jax.lax.conv_general_dilated(
        x, w, window_strides=(stride, stride), padding='SAME',
        dimension_numbers=('NHWC', 'HWIO', 'NHWC'))
    return y + b


def _conv_transpose(x, w, b, stride):
    y = jax.lax.conv_transpose(
        x, w, strides=(stride, stride), padding='SAME',
        dimension_numbers=('NHWC', 'HWIO', 'NHWC'))
    return y + b


def _rvq_kernel(z_ref, emb_ref, embt_hi_ref, embt_mid_ref, embt_lo_ref,
                zq_ref, loss_ref):
    i = pl.program_id(0)
    z = z_ref[...]
    q = jnp.zeros_like(z)
    r = z
    for layer in range(RVQ_LAYERS):
        emb = emb_ref[layer]
        sim = jnp.dot(r, emb, preferred_element_type=jnp.float32)
        rn = jnp.sum(r * r, axis=1, keepdims=True)
        en = jnp.sum(emb * emb, axis=0, keepdims=True)
        dist = rn + en - 2.0 * sim
        m = jnp.min(dist, axis=1, keepdims=True)
        iota = jax.lax.broadcasted_iota(jnp.int32, (TILE, NUM_EMBED), 1)
        cand = jnp.where(dist == m, iota, NUM_EMBED)
        idx = jnp.min(cand, axis=1, keepdims=True)
        onehot = (iota == idx).astype(jnp.float32)
        up_hi = jnp.dot(onehot, embt_hi_ref[layer],
                        preferred_element_type=jnp.float32)
        up_mid = jnp.dot(onehot, embt_mid_ref[layer],
                         preferred_element_type=jnp.float32)
        up_lo = jnp.dot(onehot, embt_lo_ref[layer],
                        preferred_element_type=jnp.float32)
        upd = (up_hi + up_mid) + up_lo
        q = q + upd
        r = r - upd
    zq_ref[...] = q
    partial = jnp.sum(r * r).reshape(1, 1)

    @pl.when(i == 0)
    def _init():
        loss_ref[...] = partial

    @pl.when(i != 0)
    def _acc():
        loss_ref[...] += partial


def _fused_rvq(z_flat, embeddings):
    n = z_flat.shape[0]
    grid = n // TILE
    emb_l = jnp.transpose(embeddings, (2, 0, 1))
    embt_l = jnp.transpose(embeddings, (2, 1, 0))
    _trunc = lambda t: jax.lax.bitcast_convert_type(
        jax.lax.bitcast_convert_type(t, jnp.uint32) & jnp.uint32(0xFFFF0000),
        jnp.float32)
    embt_hi = _trunc(embt_l)
    rem = embt_l - embt_hi
    embt_mid = _trunc(rem)
    embt_lo = rem - embt_mid

    zq, loss = pl.pallas_call(
        _rvq_kernel,
        grid=(grid,),
        in_specs=[
            pl.BlockSpec((TILE, LATENT_DIM), lambda i: (i, 0)),
            pl.BlockSpec((RVQ_LAYERS, LATENT_DIM, NUM_EMBED),
                         lambda i: (0, 0, 0)),
            pl.BlockSpec((RVQ_LAYERS, NUM_EMBED, LATENT_DIM),
                         lambda i: (0, 0, 0)),
            pl.BlockSpec((RVQ_LAYERS, NUM_EMBED, LATENT_DIM),
                         lambda i: (0, 0, 0)),
            pl.BlockSpec((RVQ_LAYERS, NUM_EMBED, LATENT_DIM),
                         lambda i: (0, 0, 0)),
        ],
        out_specs=[
            pl.BlockSpec((TILE, LATENT_DIM), lambda i: (i, 0)),
            pl.BlockSpec((1, 1), lambda i: (0, 0)),
        ],
        out_shape=[
            jax.ShapeDtypeStruct((n, LATENT_DIM), jnp.float32),
            jax.ShapeDtypeStruct((1, 1), jnp.float32),
        ],
    )(z_flat, emb_l, embt_hi, embt_mid, embt_lo)
    vq_loss = BETA * loss[0, 0] / (n * LATENT_DIM)
    return zq, vq_loss


def kernel(x, enc_w1, enc_b1, enc_w2, enc_b2, enc_w3, enc_b3,
           dec_w1, dec_b1, dec_w2, dec_b2, dec_w3, dec_b3, embeddings):
    h = jax.nn.relu(_conv(x, enc_w1, enc_b1, 2))
    h = jax.nn.relu(_conv(h, enc_w2, enc_b2, 2))
    z = _conv(h, enc_w3, enc_b3, 1)
    b, hh, ww, _ = z.shape
    zq_flat, vq_loss = _fused_rvq(z.reshape(-1, LATENT_DIM), embeddings)
    zq = zq_flat.reshape(b, hh, ww, LATENT_DIM)
    d = jax.nn.relu(_conv_transpose(zq, dec_w1, dec_b1, 2))
    d = jax.nn.relu(_conv_transpose(d, dec_w2, dec_b2, 2))
    recon = _conv_transpose(d, dec_w3, dec_b3, 1)
    return recon, vq_loss


# speedup vs baseline: 1.4126x; 1.3420x over previous
import jax
import jax.numpy as jnp
from jax.experimental import pallas as pl

LATENT_DIM = 64
NUM_EMBED = 1024
RVQ_LAYERS = 4
BETA = 0.25

TILE = 1024


def _conv(x, w, b, stride):
    y = jax.lax.conv_general_dilated(
        x, w, window_strides=(stride, stride), padding='SAME',
        dimension_numbers=('NHWC', 'HWIO', 'NHWC'))
    return y + b


def _conv_transpose(x, w, b, stride):
    y = jax.lax.conv_transpose(
        x, w, strides=(stride, stride), padding='SAME',
        dimension_numbers=('NHWC', 'HWIO', 'NHWC'))
    return y + b


NCHUNK = 2
CHUNK = TILE // NCHUNK


def _rvq_kernel(z_ref, emb_ref, embt_hi_ref, embt_mid_ref, embt_lo_ref,
                zq_ref, loss_ref):
    i = pl.program_id(0)
    iota = jax.lax.broadcasted_iota(
        jnp.int32, (CHUNK, NUM_EMBED), 1).astype(jnp.float32)

    rs = [z_ref[pl.ds(c * CHUNK, CHUNK), :] for c in range(NCHUNK)]
    qs = [jnp.zeros_like(rs[0]) for _ in range(NCHUNK)]
    for layer in range(RVQ_LAYERS):
        emb = emb_ref[layer]
        en = jnp.sum(emb * emb, axis=0, keepdims=True)
        e_hi = embt_hi_ref[layer]
        e_mid = embt_mid_ref[layer]
        e_lo = embt_lo_ref[layer]
        for c in range(NCHUNK):
            r = rs[c]
            sim = jnp.dot(r, emb, preferred_element_type=jnp.float32)
            rn = jnp.sum(r * r, axis=1, keepdims=True)
            dist = rn + en - 2.0 * sim
            m = jnp.min(dist, axis=1, keepdims=True)
            cand = jnp.where(dist == m, iota, jnp.float32(NUM_EMBED))
            idx = jnp.min(cand, axis=1, keepdims=True)
            onehot = (iota == idx).astype(jnp.float32)
            up_hi = jnp.dot(onehot, e_hi,
                            preferred_element_type=jnp.float32)
            up_mid = jnp.dot(onehot, e_mid,
                             preferred_element_type=jnp.float32)
            up_lo = jnp.dot(onehot, e_lo,
                            preferred_element_type=jnp.float32)
            upd = (up_hi + up_mid) + up_lo
            qs[c] = qs[c] + upd
            rs[c] = r - upd
    for c in range(NCHUNK):
        zq_ref[pl.ds(c * CHUNK, CHUNK), :] = qs[c]
    partial = sum(jnp.sum(r * r) for r in rs).reshape(1, 1)

    @pl.when(i == 0)
    def _init():
        loss_ref[...] = partial

    @pl.when(i != 0)
    def _acc():
        loss_ref[...] += partial


def _fused_rvq(z_flat, embeddings):
    n = z_flat.shape[0]
    grid = n // TILE
    emb_l = jnp.transpose(embeddings, (2, 0, 1))
    embt_l = jnp.transpose(embeddings, (2, 1, 0))
    _trunc = lambda t: jax.lax.bitcast_convert_type(
        jax.lax.bitcast_convert_type(t, jnp.uint32) & jnp.uint32(0xFFFF0000),
        jnp.float32)
    embt_hi = _trunc(embt_l)
    rem = embt_l - embt_hi
    embt_mid = _trunc(rem)
    embt_lo = rem - embt_mid

    zq, loss = pl.pallas_call(
        _rvq_kernel,
        grid=(grid,),
        in_specs=[
            pl.BlockSpec((TILE, LATENT_DIM), lambda i: (i, 0)),
            pl.BlockSpec((RVQ_LAYERS, LATENT_DIM, NUM_EMBED),
                         lambda i: (0, 0, 0)),
            pl.BlockSpec((RVQ_LAYERS, NUM_EMBED, LATENT_DIM),
                         lambda i: (0, 0, 0)),
            pl.BlockSpec((RVQ_LAYERS, NUM_EMBED, LATENT_DIM),
                         lambda i: (0, 0, 0)),
            pl.BlockSpec((RVQ_LAYERS, NUM_EMBED, LATENT_DIM),
                         lambda i: (0, 0, 0)),
        ],
        out_specs=[
            pl.BlockSpec((TILE, LATENT_DIM), lambda i: (i, 0)),
            pl.BlockSpec((1, 1), lambda i: (0, 0)),
        ],
        out_shape=[
            jax.ShapeDtypeStruct((n, LATENT_DIM), jnp.float32),
            jax.ShapeDtypeStruct((1, 1), jnp.float32),
        ],
    )(z_flat, emb_l, embt_hi, embt_mid, embt_lo)
    vq_loss = BETA * loss[0, 0] / (n * LATENT_DIM)
    return zq, vq_loss


def kernel(x, enc_w1, enc_b1, enc_w2, enc_b2, enc_w3, enc_b3,
           dec_w1, dec_b1, dec_w2, dec_b2, dec_w3, dec_b3, embeddings):
    h = jax.nn.relu(_conv(x, enc_w1, enc_b1, 2))
    h = jax.nn.relu(_conv(h, enc_w2, enc_b2, 2))
    z = _conv(h, enc_w3, enc_b3, 1)
    b, hh, ww, _ = z.shape
    zq_flat, vq_loss = _fused_rvq(z.reshape(-1, LATENT_DIM), embeddings)
    zq = zq_flat.reshape(b, hh, ww, LATENT_DIM)
    d = jax.nn.relu(_conv_transpose(zq, dec_w1, dec_b1, 2))
    d = jax.nn.relu(_conv_transpose(d, dec_w2, dec_b2, 2))
    recon = _conv_transpose(d, dec_w3, dec_b3, 1)
    return recon, vq_loss


# speedup vs baseline: 1.4833x; 1.0500x over previous
import jax
import jax.numpy as jnp
from jax.experimental import pallas as pl

LATENT_DIM = 64
NUM_EMBED = 1024
RVQ_LAYERS = 4
BETA = 0.25

TILE = 1024


def _conv(x, w, b, stride):
    y = jax.lax.conv_general_dilated(
        x, w, window_strides=(stride, stride), padding='SAME',
        dimension_numbers=('NHWC', 'HWIO', 'NHWC'))
    return y + b


def _conv_transpose(x, w, b, stride):
    y = jax.lax.conv_transpose(
        x, w, strides=(stride, stride), padding='SAME',
        dimension_numbers=('NHWC', 'HWIO', 'NHWC'))
    return y + b


NCHUNK = 4
CHUNK = TILE // NCHUNK


def _rvq_kernel(z_ref, emb_ref, en_ref, zq_ref, loss_ref):
    i = pl.program_id(0)
    iota = jax.lax.broadcasted_iota(
        jnp.int32, (CHUNK, NUM_EMBED), 1).astype(jnp.float32)

    rs = [z_ref[pl.ds(c * CHUNK, CHUNK), :] for c in range(NCHUNK)]
    qs = [jnp.zeros_like(rs[0]) for _ in range(NCHUNK)]
    for layer in range(RVQ_LAYERS):
        emb = emb_ref[layer]
        en = en_ref[pl.ds(layer, 1), :]
        for c in range(NCHUNK):
            r = rs[c]
            sim = jnp.dot(r, emb, preferred_element_type=jnp.float32)
            rn = jnp.sum(r * r, axis=1, keepdims=True)
            dist = rn + en - 2.0 * sim
            m = jnp.min(dist, axis=1, keepdims=True)
            cand = jnp.where(dist == m, iota, jnp.float32(NUM_EMBED))
            idx = jnp.min(cand, axis=1, keepdims=True)
            idx_lane = jnp.broadcast_to(
                idx.astype(jnp.int32).T, (LATENT_DIM, CHUNK))
            low = idx_lane & 127
            hi = idx_lane >> 7
            updT = jnp.zeros((LATENT_DIM, CHUNK), jnp.float32)
            for j in range(NUM_EMBED // 128):
                gj = jnp.take_along_axis(
                    emb[:, j * 128:(j + 1) * 128], low, axis=1)
                updT = jnp.where(hi == j, gj, updT)
            upd = updT.T
            qs[c] = qs[c] + upd
            rs[c] = r - upd
    for c in range(NCHUNK):
        zq_ref[pl.ds(c * CHUNK, CHUNK), :] = qs[c]
    partial = sum(jnp.sum(r * r) for r in rs).reshape(1, 1)

    @pl.when(i == 0)
    def _init():
        loss_ref[...] = partial

    @pl.when(i != 0)
    def _acc():
        loss_ref[...] += partial


def _fused_rvq(z_flat, embeddings):
    n = z_flat.shape[0]
    grid = n // TILE
    emb_l = jnp.transpose(embeddings, (2, 0, 1))
    en_l = jnp.sum(embeddings * embeddings, axis=0).T

    zq, loss = pl.pallas_call(
        _rvq_kernel,
        grid=(grid,),
        in_specs=[
            pl.BlockSpec((TILE, LATENT_DIM), lambda i: (i, 0)),
            pl.BlockSpec((RVQ_LAYERS, LATENT_DIM, NUM_EMBED),
                         lambda i: (0, 0, 0)),
            pl.BlockSpec((RVQ_LAYERS, NUM_EMBED), lambda i: (0, 0)),
        ],
        out_specs=[
            pl.BlockSpec((TILE, LATENT_DIM), lambda i: (i, 0)),
            pl.BlockSpec((1, 1), lambda i: (0, 0)),
        ],
        out_shape=[
            jax.ShapeDtypeStruct((n, LATENT_DIM), jnp.float32),
            jax.ShapeDtypeStruct((1, 1), jnp.float32),
        ],
    )(z_flat, emb_l, en_l)
    vq_loss = BETA * loss[0, 0] / (n * LATENT_DIM)
    return zq, vq_loss


def kernel(x, enc_w1, enc_b1, enc_w2, enc_b2, enc_w3, enc_b3,
           dec_w1, dec_b1, dec_w2, dec_b2, dec_w3, dec_b3, embeddings):
    h = jax.nn.relu(_conv(x, enc_w1, enc_b1, 2))
    h = jax.nn.relu(_conv(h, enc_w2, enc_b2, 2))
    z = _conv(h, enc_w3, enc_b3, 1)
    b, hh, ww, _ = z.shape
    zq_flat, vq_loss = _fused_rvq(z.reshape(-1, LATENT_DIM), embeddings)
    zq = zq_flat.reshape(b, hh, ww, LATENT_DIM)
    d = jax.nn.relu(_conv_transpose(zq, dec_w1, dec_b1, 2))
    d = jax.nn.relu(_conv_transpose(d, dec_w2, dec_b2, 2))
    recon = _conv_transpose(d, dec_w3, dec_b3, 1)
    return recon, vq_loss
